# Initial kernel scaffold; baseline (speedup 1.0000x reference)
import functools

import jax
import jax.numpy as jnp
from jax.experimental import pallas as pl
from jax.experimental.pallas import tpu as pltpu

E = 8
TM = 1024


def _moe_body(x_ref, gw_ref, gb_ref, w_ref, b_ref, out_ref, logits_ref, wts_ref):
    e = pl.program_id(1)

    @pl.when(e == 0)
    def _router():
        xs = x_ref[...]
        logits = jax.lax.dot_general(
            xs, gw_ref[...], (((1,), (1,)), ((), ())),
            preferred_element_type=jnp.float32) + gb_ref[...]
        logits_ref[...] = logits
        probs = jax.nn.softmax(logits, axis=-1)
        iota = jax.lax.broadcasted_iota(jnp.int32, probs.shape, 1)
        a1 = jnp.argmax(probs, axis=-1, keepdims=True)
        m1 = jnp.max(probs, axis=-1, keepdims=True)
        probs2 = jnp.where(iota == a1, -jnp.inf, probs)
        a2 = jnp.argmax(probs2, axis=-1, keepdims=True)
        m2 = jnp.max(probs2, axis=-1, keepdims=True)
        sel = (iota == a1) | (iota == a2)
        wts_ref[...] = jnp.where(sel, probs, 0.0) / (m1 + m2)

    xb = x_ref[...].astype(jnp.bfloat16)
    mm = jax.lax.dot_general(
        xb, w_ref[0], (((1,), (1,)), ((), ())),
        preferred_element_type=jnp.float32)
    iota = jax.lax.broadcasted_iota(jnp.int32, wts_ref.shape, 1)
    w_col = jnp.sum(wts_ref[...] * (iota == e), axis=1, keepdims=True)
    contrib = (mm + b_ref[...]) * w_col

    @pl.when(e == 0)
    def _init():
        out_ref[...] = contrib

    @pl.when(e != 0)
    def _acc():
        out_ref[...] += contrib


@jax.jit
def kernel(x, gate_W, gate_b, W, b):
    Bx, Sx, Hx = x.shape
    T = Bx * Sx
    hs = x.reshape(T, Hx)
    W_bf = W.astype(jnp.bfloat16)
    gb2 = gate_b.reshape(1, E)

    grid = (T // TM, E)
    out, logits = pl.pallas_call(
        _moe_body,
        grid=grid,
        in_specs=[
            pl.BlockSpec((TM, Hx), lambda m, e: (m, 0)),
            pl.BlockSpec((E, Hx), lambda m, e: (0, 0)),
            pl.BlockSpec((1, E), lambda m, e: (0, 0)),
            pl.BlockSpec((1, Hx, Hx), lambda m, e: (e, 0, 0)),
            pl.BlockSpec((1, Hx), lambda m, e: (e, 0)),
        ],
        out_specs=[
            pl.BlockSpec((TM, Hx), lambda m, e: (m, 0)),
            pl.BlockSpec((TM, E), lambda m, e: (m, 0)),
        ],
        out_shape=[
            jax.ShapeDtypeStruct((T, Hx), jnp.float32),
            jax.ShapeDtypeStruct((T, E), jnp.float32),
        ],
        scratch_shapes=[pltpu.VMEM((TM, E), jnp.float32)],
        compiler_params=pltpu.CompilerParams(
            dimension_semantics=("parallel", "arbitrary"),
        ),
    )(hs, gate_W, gb2, W_bf, b)
    return out.reshape(Bx, Sx, Hx), logits


# speedup vs baseline: 1.1322x; 1.1322x over previous
import functools

import jax
import jax.numpy as jnp
from jax.experimental import pallas as pl
from jax.experimental.pallas import tpu as pltpu

E = 8
TM = 512


def _moe_body(x_ref, gw_ref, gb_ref, w_ref, b_ref, out_ref, logits_ref, wts_ref):
    e = pl.program_id(1)

    @pl.when(e == 0)
    def _router():
        xs = x_ref[...]
        logits = jax.lax.dot_general(
            xs, gw_ref[...], (((1,), (1,)), ((), ())),
            preferred_element_type=jnp.float32) + gb_ref[...]
        logits_ref[...] = logits
        probs = jax.nn.softmax(logits, axis=-1)
        iota = jax.lax.broadcasted_iota(jnp.int32, probs.shape, 1)
        a1 = jnp.argmax(probs, axis=-1, keepdims=True)
        m1 = jnp.max(probs, axis=-1, keepdims=True)
        probs2 = jnp.where(iota == a1, -jnp.inf, probs)
        a2 = jnp.argmax(probs2, axis=-1, keepdims=True)
        m2 = jnp.max(probs2, axis=-1, keepdims=True)
        sel = (iota == a1) | (iota == a2)
        wts_ref[...] = jnp.where(sel, probs, 0.0) / (m1 + m2)

    xb = x_ref[...].astype(jnp.bfloat16)
    mm = jax.lax.dot_general(
        xb, w_ref[0], (((1,), (1,)), ((), ())),
        preferred_element_type=jnp.float32)
    iota = jax.lax.broadcasted_iota(jnp.int32, wts_ref.shape, 1)
    w_col = jnp.sum(wts_ref[...] * (iota == e), axis=1, keepdims=True)
    contrib = (mm + b_ref[0]) * w_col

    @pl.when(e == 0)
    def _init():
        out_ref[...] = contrib

    @pl.when(e != 0)
    def _acc():
        out_ref[...] += contrib


@jax.jit
def kernel(x, gate_W, gate_b, W, b):
    Bx, Sx, Hx = x.shape
    T = Bx * Sx
    hs = x.reshape(T, Hx)
    W_bf = W.astype(jnp.bfloat16)
    gb2 = gate_b.reshape(1, E)
    b3 = b.reshape(E, 1, Hx)

    grid = (T // TM, E)
    out, logits = pl.pallas_call(
        _moe_body,
        grid=grid,
        in_specs=[
            pl.BlockSpec((TM, Hx), lambda m, e: (m, 0)),
            pl.BlockSpec((E, Hx), lambda m, e: (0, 0)),
            pl.BlockSpec((1, E), lambda m, e: (0, 0)),
            pl.BlockSpec((1, Hx, Hx), lambda m, e: (e, 0, 0)),
            pl.BlockSpec((1, 1, Hx), lambda m, e: (e, 0, 0)),
        ],
        out_specs=[
            pl.BlockSpec((TM, Hx), lambda m, e: (m, 0)),
            pl.BlockSpec((TM, E), lambda m, e: (m, 0)),
        ],
        out_shape=[
            jax.ShapeDtypeStruct((T, Hx), jnp.float32),
            jax.ShapeDtypeStruct((T, E), jnp.float32),
        ],
        scratch_shapes=[pltpu.VMEM((TM, E), jnp.float32)],
        compiler_params=pltpu.CompilerParams(
            dimension_semantics=("parallel", "arbitrary"),
        ),
    )(hs, gate_W, gb2, W_bf, b3)
    return out.reshape(Bx, Sx, Hx), logits
